# Initial kernel scaffold; baseline (speedup 1.0000x reference)
import jax
import jax.numpy as jnp
from jax.experimental import pallas as pl


def kernel(input):
    raise NotImplementedError("write your pallas kernel here")


# speedup vs baseline: 4.9939x; 4.9939x over previous
import functools

import jax
import jax.numpy as jnp
from jax import lax
from jax.experimental import pallas as pl
from jax.experimental.pallas import tpu as pltpu
from jax.experimental.pallas import tpu_sc as plsc

F = 59
C = F * F
HALF = 29
L = 16
OFFS = (0, 16, 32, 43)


def _build(n_batch: int):
    nb = n_batch * C
    mesh = plsc.VectorSubcoreMesh(core_axis_name="c", subcore_axis_name="s")
    nw = mesh.num_cores * mesh.num_subcores
    per_w = (nb + nw - 1) // nw

    @functools.partial(
        pl.kernel,
        out_type=jax.ShapeDtypeStruct((n_batch * C, F, F), jnp.float32),
        mesh=mesh,
        scratch_types=[
            pltpu.VMEM((F, F), jnp.float32),
            pltpu.VMEM((F, F), jnp.float32),
            pltpu.VMEM((F, F), jnp.float32),
        ],
        compiler_params=pltpu.CompilerParams(needs_layout_passes=False),
    )
    def psamask_sc(x_hbm, out_hbm, inb, outb, zb):
        wid = lax.axis_index("s") * mesh.num_cores + lax.axis_index("c")
        iota16 = lax.iota(jnp.int32, L)
        zeros16 = jnp.zeros((L,), jnp.float32)

        def zero_row(b, carry):
            for off in OFFS:
                zb[b, pl.ds(off, L)] = zeros16
            return carry

        lax.fori_loop(0, F, zero_row, 0)

        def shear_row(b, carry):
            for off in OFFS:
                w = iota16 + off
                d = (b + HALF - off) - iota16
                valid = (d >= 0) & (d < F)
                rbv = jnp.where(valid, d, 0)
                gv = plsc.load_gather(inb, [rbv, w])
                gv = jnp.where(valid, gv, 0.0)
                outb[b, pl.ds(off, L)] = gv
            return carry

        def block_loop(t, carry):
            g = wid + nw * t

            @pl.when(g < nb)
            def _():
                n = g // C
                r = g % C
                a = r // F
                h = r % F
                d = a + HALF - h
                in_band = (d >= 0) & (d < F)
                out_slice = out_hbm.at[pl.ds(n * C + F * a, F), h]

                @pl.when(in_band)
                def _():
                    pltpu.sync_copy(x_hbm.at[pl.ds(n * C + F * d, F), h], inb)
                    lax.fori_loop(0, F, shear_row, 0)
                    pltpu.sync_copy(outb, out_slice)

                @pl.when(jnp.logical_not(in_band))
                def _():
                    pltpu.sync_copy(zb, out_slice)

            return carry

        lax.fori_loop(0, per_w, block_loop, 0)

    return psamask_sc


def kernel(input):
    n_batch = input.shape[0]
    x = input.reshape(n_batch * C, F, F)
    out = _build(n_batch)(x)
    return out.reshape(n_batch, C, F, F)


# speedup vs baseline: 5.9797x; 1.1974x over previous
import functools

import jax
import jax.numpy as jnp
from jax import lax
from jax.experimental import pallas as pl
from jax.experimental.pallas import tpu as pltpu
from jax.experimental.pallas import tpu_sc as plsc

F = 59
C = F * F
HALF = 29
L = 16
OFFS = (0, 16, 32, 43)


def _build(n_batch: int):
    nb = n_batch * C
    mesh = plsc.VectorSubcoreMesh(core_axis_name="c", subcore_axis_name="s")
    nw = mesh.num_cores * mesh.num_subcores
    per_w = (nb + nw - 1) // nw
    assert per_w % 2 == 0

    @functools.partial(
        pl.kernel,
        out_type=jax.ShapeDtypeStruct((n_batch * C, F, F), jnp.float32),
        mesh=mesh,
        scratch_types=[
            pltpu.VMEM((F, F), jnp.float32),
            pltpu.VMEM((F, F), jnp.float32),
            pltpu.VMEM((F, F), jnp.float32),
            pltpu.VMEM((F, F), jnp.float32),
            pltpu.VMEM((F, F), jnp.float32),
            pltpu.SemaphoreType.DMA,
            pltpu.SemaphoreType.DMA,
            pltpu.SemaphoreType.DMA,
            pltpu.SemaphoreType.DMA,
            pltpu.SemaphoreType.DMA,
        ],
        compiler_params=pltpu.CompilerParams(needs_layout_passes=False),
    )
    def psamask_sc(x_hbm, out_hbm, inb0, inb1, outb0, outb1, zb,
                   in_sem0, in_sem1, out_sem0, out_sem1, zsem):
        wid = lax.axis_index("s") * mesh.num_cores + lax.axis_index("c")
        iota16 = lax.iota(jnp.int32, L)
        zeros16 = jnp.zeros((L,), jnp.float32)

        def zero_row(b, carry):
            for off in OFFS:
                zb[b, pl.ds(off, L)] = zeros16
            return carry

        lax.fori_loop(0, F, zero_row, 0)

        def shear(inb, outb):
            for b in range(F):
                for off in OFFS:
                    wv = iota16 + off
                    d = (b + HALF - off) - iota16
                    valid = (d >= 0) & (d < F)
                    rbv = jnp.where(valid, d, 0)
                    gv = plsc.load_gather(inb, [rbv, wv])
                    gv = jnp.where(valid, gv, 0.0)
                    outb[b, pl.ds(off, L)] = gv

        def decode(t):
            g = wid + nw * t
            n = g // C
            r = g % C
            a = r // F
            h = r % F
            d = a + HALF - h
            in_band = (g < nb) & (d >= 0) & (d < F)
            oob = (g < nb) & jnp.logical_not(in_band)
            return n, a, h, d, in_band, oob

        def issue_in(t, inb, in_sem):
            n, a, h, d, in_band, oob = decode(t)

            @pl.when(in_band)
            def _():
                pltpu.async_copy(
                    x_hbm.at[pl.ds(n * C + F * d, F), h], inb, in_sem)

        def slot(t, inb, outb, in_sem, out_sem, po, zcnt,
                 inb_next, in_sem_next):
            n, a, h, d, in_band, oob = decode(t)
            issue_in(t + 1, inb_next, in_sem_next)
            out_slice = out_hbm.at[pl.ds(n * C + F * a, F), h]

            @pl.when(in_band)
            def _():
                pltpu.make_async_copy(
                    x_hbm.at[pl.ds(0, F), 0], inb, in_sem).wait()

                @pl.when(po > 0)
                def _():
                    pltpu.make_async_copy(
                        outb, out_hbm.at[pl.ds(0, F), 0], out_sem).wait()

                shear(inb, outb)
                pltpu.async_copy(outb, out_slice, out_sem)

            @pl.when(oob)
            def _():
                pltpu.async_copy(zb, out_slice, zsem)

            po = jnp.where(in_band, jnp.int32(1), po)
            zcnt = zcnt + oob.astype(jnp.int32)
            return po, zcnt

        issue_in(0, inb0, in_sem0)

        def uloop(u, carry):
            po0, po1, zcnt = carry
            po0, zcnt = slot(2 * u, inb0, outb0, in_sem0, out_sem0,
                             po0, zcnt, inb1, in_sem1)
            po1, zcnt = slot(2 * u + 1, inb1, outb1, in_sem1, out_sem1,
                             po1, zcnt, inb0, in_sem0)
            return po0, po1, zcnt

        po0, po1, zcnt = lax.fori_loop(
            0, per_w // 2, uloop,
            (jnp.int32(0), jnp.int32(0), jnp.int32(0)))

        @pl.when(po0 > 0)
        def _():
            pltpu.make_async_copy(
                outb0, out_hbm.at[pl.ds(0, F), 0], out_sem0).wait()

        @pl.when(po1 > 0)
        def _():
            pltpu.make_async_copy(
                outb1, out_hbm.at[pl.ds(0, F), 0], out_sem1).wait()

        def zdrain(i, carry):
            pltpu.make_async_copy(
                zb, out_hbm.at[pl.ds(0, F), 0], zsem).wait()
            return carry

        lax.fori_loop(0, zcnt, zdrain, 0)

    return psamask_sc


def kernel(input):
    n_batch = input.shape[0]
    x = input.reshape(n_batch * C, F, F)
    out = _build(n_batch)(x)
    return out.reshape(n_batch, C, F, F)


# speedup vs baseline: 6.5419x; 1.0940x over previous
import functools

import jax
import jax.numpy as jnp
from jax import lax
from jax.experimental import pallas as pl
from jax.experimental.pallas import tpu as pltpu
from jax.experimental.pallas import tpu_sc as plsc

F = 59
C = F * F
HALF = 29
L = 16
OFFS = (0, 16, 32, 43)
RING = 6
PH = 64
PW = 128


def _build(n_batch: int):
    nb = n_batch * C
    mesh = plsc.VectorSubcoreMesh(core_axis_name="c", subcore_axis_name="s")
    nw = mesh.num_cores * mesh.num_subcores
    per_w = (nb + nw - 1) // nw
    n_iter = (per_w + RING - 1) // RING

    @functools.partial(
        pl.kernel,
        out_type=jax.ShapeDtypeStruct((n_batch * C, PH, PW), jnp.float32),
        mesh=mesh,
        scratch_types=(
            [pltpu.VMEM((F, F), jnp.float32)] * RING
            + [pltpu.VMEM((F, PW), jnp.float32)] * (RING + 1)
            + [pltpu.SemaphoreType.DMA] * (2 * RING + 1)
        ),
        compiler_params=pltpu.CompilerParams(needs_layout_passes=False),
    )
    def psamask_sc(x_hbm, out_hbm, *scr):
        inb = scr[:RING]
        outb = scr[RING:2 * RING]
        zb = scr[2 * RING]
        isem = scr[2 * RING + 1:3 * RING + 1]
        osem = scr[3 * RING + 1:4 * RING + 1]
        zsem = scr[4 * RING + 1]

        wid = lax.axis_index("s") * mesh.num_cores + lax.axis_index("c")
        iota16 = lax.iota(jnp.int32, L)
        zeros16 = jnp.zeros((L,), jnp.float32)

        def zero_row(b, carry):
            for off in range(0, PW, L):
                zb[b, pl.ds(off, L)] = zeros16
            return carry

        lax.fori_loop(0, F, zero_row, 0)

        def shear(src, dst):
            def row(b, carry):
                for off in OFFS:
                    wv = iota16 + off
                    d = (b + HALF - off) - iota16
                    valid = (d >= 0) & (d < F)
                    rbv = jnp.where(valid, d, 0)
                    gv = plsc.load_gather(src, [rbv, wv])
                    gv = jnp.where(valid, gv, 0.0)
                    dst[b, pl.ds(off, L)] = gv
                return carry

            lax.fori_loop(0, F, row, 0)

        def decode(t):
            g = wid + nw * t
            n = g // C
            r = g % C
            a = r // F
            h = r % F
            d = a + HALF - h
            in_band = (g < nb) & (d >= 0) & (d < F)
            oob = (g < nb) & jnp.logical_not(in_band)
            return n, a, h, d, in_band, oob

        def issue_in(t, e):
            n, a, h, d, in_band, oob = decode(t)

            @pl.when(in_band)
            def _():
                pltpu.async_copy(
                    x_hbm.at[pl.ds(n * C + F * d, F), h], inb[e], isem[e])

        def slot(t, e, po, zcnt):
            n, a, h, d, in_band, oob = decode(t)
            out_slice = out_hbm.at[pl.ds(n * C + F * a, F), h]

            @pl.when(in_band)
            def _():
                pltpu.make_async_copy(
                    x_hbm.at[pl.ds(0, F), 0], inb[e], isem[e]).wait()

                @pl.when(po > 0)
                def _():
                    pltpu.make_async_copy(
                        outb[e], out_hbm.at[pl.ds(0, F), 0], osem[e]).wait()

                shear(inb[e], outb[e])
                pltpu.async_copy(outb[e], out_slice, osem[e])

            @pl.when(oob)
            def _():
                pltpu.async_copy(zb, out_slice, zsem)

            issue_in(t + RING, e)
            po = jnp.where(in_band, jnp.int32(1), po)
            zcnt = zcnt + oob.astype(jnp.int32)
            return po, zcnt

        for e in range(RING):
            issue_in(e, e)

        def vloop(v, carry):
            pos = list(carry[:RING])
            zcnt = carry[RING]
            for e in range(RING):
                pos[e], zcnt = slot(RING * v + e, e, pos[e], zcnt)
            return tuple(pos) + (zcnt,)

        carry = lax.fori_loop(
            0, n_iter, vloop, (jnp.int32(0),) * RING + (jnp.int32(0),))

        for e in range(RING):

            @pl.when(carry[e] > 0)
            def _(e=e):
                pltpu.make_async_copy(
                    outb[e], out_hbm.at[pl.ds(0, F), 0], osem[e]).wait()

        def zdrain(i, c):
            pltpu.make_async_copy(
                zb, out_hbm.at[pl.ds(0, F), 0], zsem).wait()
            return c

        lax.fori_loop(0, carry[RING], zdrain, 0)

    return psamask_sc


def kernel(input):
    n_batch = input.shape[0]
    x = input.reshape(n_batch * C, F, F)
    padded = _build(n_batch)(x)
    return padded[:, :F, :F].reshape(n_batch, C, F, F)


# speedup vs baseline: 8.2746x; 1.2649x over previous
import functools

import jax
import jax.numpy as jnp
from jax import lax
from jax.experimental import pallas as pl
from jax.experimental.pallas import tpu as pltpu
from jax.experimental.pallas import tpu_sc as plsc

F = 59
C = F * F
HALF = 29
L = 16
OFFS = (0, 16, 32, 43)
RING = 6


def _build(n_batch: int):
    nb = n_batch * C
    mesh = plsc.VectorSubcoreMesh(core_axis_name="c", subcore_axis_name="s")
    nw = mesh.num_cores * mesh.num_subcores
    per_w = (nb + nw - 1) // nw
    n_iter = (per_w + RING - 1) // RING

    @functools.partial(
        pl.kernel,
        out_type=jax.ShapeDtypeStruct((n_batch, C, F, F), jnp.float32),
        mesh=mesh,
        scratch_types=(
            [pltpu.VMEM((F, F), jnp.float32)] * RING
            + [pltpu.VMEM((F, F), jnp.float32)] * (RING + 1)
            + [pltpu.SemaphoreType.DMA] * (2 * RING + 1)
        ),
        compiler_params=pltpu.CompilerParams(needs_layout_passes=False),
    )
    def psamask_sc(x_hbm, out_hbm, *scr):
        inb = scr[:RING]
        outb = scr[RING:2 * RING]
        zb = scr[2 * RING]
        isem = scr[2 * RING + 1:3 * RING + 1]
        osem = scr[3 * RING + 1:4 * RING + 1]
        zsem = scr[4 * RING + 1]

        wid = lax.axis_index("s") * mesh.num_cores + lax.axis_index("c")
        iota16 = lax.iota(jnp.int32, L)
        zeros16 = jnp.zeros((L,), jnp.float32)

        def zero_row(b, carry):
            for off in OFFS:
                zb[b, pl.ds(off, L)] = zeros16
            return carry

        lax.fori_loop(0, F, zero_row, 0)

        def shear(src, dst):
            def row(b, carry):
                for off in OFFS:
                    wv = iota16 + off
                    d = (b + HALF - off) - iota16
                    valid = (d >= 0) & (d < F)
                    rbv = jnp.where(valid, d, 0)
                    gv = plsc.load_gather(src, [rbv, wv])
                    gv = jnp.where(valid, gv, 0.0)
                    dst[b, pl.ds(off, L)] = gv
                return carry

            lax.fori_loop(0, F, row, 0)

        def decode(t):
            g = wid + nw * t
            n = g // C
            r = g % C
            a = r // F
            h = r % F
            d = a + HALF - h
            in_band = (g < nb) & (d >= 0) & (d < F)
            oob = (g < nb) & jnp.logical_not(in_band)
            return n, a, h, d, in_band, oob

        def issue_in(t, e):
            n, a, h, d, in_band, oob = decode(t)

            @pl.when(in_band)
            def _():
                pltpu.async_copy(
                    x_hbm.at[n, pl.ds(F * d, F), h], inb[e], isem[e])

        def slot(t, e, po, zcnt):
            n, a, h, d, in_band, oob = decode(t)
            out_slice = out_hbm.at[n, pl.ds(F * a, F), h]

            @pl.when(in_band)
            def _():
                pltpu.make_async_copy(
                    x_hbm.at[0, pl.ds(0, F), 0], inb[e], isem[e]).wait()

                @pl.when(po > 0)
                def _():
                    pltpu.make_async_copy(
                        outb[e], out_hbm.at[0, pl.ds(0, F), 0], osem[e]).wait()

                shear(inb[e], outb[e])
                pltpu.async_copy(outb[e], out_slice, osem[e])

            @pl.when(oob)
            def _():
                pltpu.async_copy(zb, out_slice, zsem)

            issue_in(t + RING, e)
            po = jnp.where(in_band, jnp.int32(1), po)
            zcnt = zcnt + oob.astype(jnp.int32)
            return po, zcnt

        for e in range(RING):
            issue_in(e, e)

        def vloop(v, carry):
            pos = list(carry[:RING])
            zcnt = carry[RING]
            for e in range(RING):
                pos[e], zcnt = slot(RING * v + e, e, pos[e], zcnt)
            return tuple(pos) + (zcnt,)

        carry = lax.fori_loop(
            0, n_iter, vloop, (jnp.int32(0),) * RING + (jnp.int32(0),))

        for e in range(RING):

            @pl.when(carry[e] > 0)
            def _(e=e):
                pltpu.make_async_copy(
                    outb[e], out_hbm.at[0, pl.ds(0, F), 0], osem[e]).wait()

        def zdrain(i, c):
            pltpu.make_async_copy(
                zb, out_hbm.at[0, pl.ds(0, F), 0], zsem).wait()
            return c

        lax.fori_loop(0, carry[RING], zdrain, 0)

    return psamask_sc


def kernel(input):
    n_batch = input.shape[0]
    return _build(n_batch)(input)
